# baseline (speedup 1.0000x reference)
import jax
import jax.numpy as jnp
from jax.experimental import pallas as pl
from jax.experimental.pallas import tpu as pltpu
from jax.experimental.pallas import tpu_sc as plsc

B = 16384
NUM_CLASSES = 100
WORDS_PER_CLASS = 3
TOPK = 5
VOCAB = 100000
GLOVE_D = 300
FEAT = 1236
D_OUT = 1024

BLK = 1024


NW = 16
KPW = 24
ACC_ROWS = 128
TRASH = 127


def _sc_gather_body(idx_hbm, tgt_hbm, w2v_hbm, zeros_hbm, out_hbm,
                    idx_v, tgt_v, rows_v, acc, sem):
    c = jax.lax.axis_index("c")
    s = jax.lax.axis_index("s")

    @pl.when(c == 0)
    def _():
        pltpu.sync_copy(zeros_hbm, acc.at[pl.ds(s * 8, 8)])
        plsc.subcore_barrier()
        pltpu.sync_copy(idx_hbm.at[s], idx_v)
        pltpu.sync_copy(tgt_hbm.at[s], tgt_v)
        pltpu.async_copy(w2v_hbm.at[idx_v], rows_v, sem).wait()
        pltpu.sync_copy(rows_v, acc.at[tgt_v], add=True)
        plsc.subcore_barrier()

        @pl.when(s < NUM_CLASSES // 8)
        def _():
            pltpu.sync_copy(acc.at[pl.ds(s * 8, 8)], out_hbm.at[pl.ds(s * 8, 8)])

        @pl.when(s == NUM_CLASSES // 8)
        def _():
            pltpu.sync_copy(acc.at[pl.ds(96, NUM_CLASSES - 96)],
                            out_hbm.at[pl.ds(96, NUM_CLASSES - 96)])


NIDX = NUM_CLASSES * WORDS_PER_CLASS


def _gather_body(grp_ref, sel_ref, w2v_ref, table_ref, buf_ref, sem_ref):
    for j in range(NIDX):
        g = grp_ref[j]
        pltpu.make_async_copy(
            w2v_ref.at[pl.ds(g * 8, 8), :], buf_ref.at[pl.ds(j * 8, 8), :],
            sem_ref.at[j]).start()
    for j in range(NIDX):
        pltpu.make_async_copy(
            w2v_ref.at[pl.ds(0, 8), :], buf_ref.at[pl.ds(j * 8, 8), :],
            sem_ref.at[j]).wait()
    table_ref[...] = jnp.dot(sel_ref[...], buf_ref[...],
                             preferred_element_type=jnp.float32)


def _main_body(ce_ref, lf_ref, table_ref, w1_ref, w2_ref, b_ref,
               out_ref, cls_out_ref):
    i = pl.program_id(0)

    @pl.when(i == 0)
    def _():
        cls_out_ref[...] = (
            jnp.dot(table_ref[...], w2_ref[...],
                    preferred_element_type=jnp.float32)
            + b_ref[...]
        )

    ce = ce_ref[...]
    iota = jax.lax.broadcasted_iota(jnp.int32, (BLK, NUM_CLASSES), 1)
    counts = jnp.zeros((BLK, NUM_CLASSES), jnp.float32)
    for k in range(TOPK):
        counts += (ce[:, k][:, None] == iota).astype(jnp.float32)
    counts = counts * (1.0 / TOPK)
    out_ref[...] = (
        jnp.dot(lf_ref[...], w1_ref[...], preferred_element_type=jnp.float32)
        + jnp.dot(counts, cls_out_ref[...], preferred_element_type=jnp.float32)
    )


def kernel(layers_feature, classes_embed, class_word_indices, word2vec, W, b):
    wi = class_word_indices.reshape(-1)
    grp = (wi // 8).astype(jnp.int32)
    sub = wi % 8
    j = jnp.arange(NIDX, dtype=jnp.int32)
    cls_of = j // WORDS_PER_CLASS
    msk = jnp.where(wi != 0, 1.0 / WORDS_PER_CLASS, 0.0).astype(jnp.float32)
    sel = jnp.zeros((NUM_CLASSES, NIDX * 8), jnp.float32)
    sel = sel.at[cls_of, j * 8 + sub].add(msk)

    table = pl.pallas_call(
        _gather_body,
        in_specs=[
            pl.BlockSpec(memory_space=pltpu.MemorySpace.SMEM),
            pl.BlockSpec(memory_space=pltpu.MemorySpace.VMEM),
            pl.BlockSpec(memory_space=pltpu.MemorySpace.HBM),
        ],
        out_specs=pl.BlockSpec(memory_space=pltpu.MemorySpace.VMEM),
        out_shape=jax.ShapeDtypeStruct((NUM_CLASSES, GLOVE_D), jnp.float32),
        scratch_shapes=[
            pltpu.VMEM((NIDX * 8, GLOVE_D), jnp.float32),
            pltpu.SemaphoreType.DMA((NIDX,)),
        ],
    )(grp, sel, word2vec)

    W1 = W[:FEAT]
    W2 = W[FEAT:]
    b2 = b.reshape(1, D_OUT)

    out = pl.pallas_call(
        _main_body,
        grid=(B // BLK,),
        in_specs=[
            pl.BlockSpec((BLK, TOPK), lambda i: (i, 0)),
            pl.BlockSpec((BLK, FEAT), lambda i: (i, 0)),
            pl.BlockSpec((NUM_CLASSES, GLOVE_D), lambda i: (0, 0)),
            pl.BlockSpec((FEAT, D_OUT), lambda i: (0, 0)),
            pl.BlockSpec((GLOVE_D, D_OUT), lambda i: (0, 0)),
            pl.BlockSpec((1, D_OUT), lambda i: (0, 0)),
        ],
        out_specs=pl.BlockSpec((BLK, D_OUT), lambda i: (i, 0)),
        out_shape=jax.ShapeDtypeStruct((B, D_OUT), jnp.float32),
        scratch_shapes=[pltpu.VMEM((NUM_CLASSES, D_OUT), jnp.float32)],
    )(classes_embed, layers_feature, table, W1, W2, b2)
    return out
